# baseline (speedup 1.0000x reference)
import dataclasses
import functools

import jax
import jax.numpy as jnp
from jax import lax
from jax.experimental import pallas as pl
from jax.experimental.pallas import tpu as pltpu
from jax.experimental.pallas import tpu_sc as plsc

N_ATOMS = 1600000
N_MOL = 50000
NC = 2
NS = 16
L = 16
NW = NC * NS
CHUNK = N_ATOMS // NW
BLK = 1000
NBLK = CHUNK // BLK
P = 50176
PS = P // NS
CHK = 3136
P2 = P + CHK + 64
PS2 = P2 // NS

_GATHER_DNUMS = jax.lax.GatherDimensionNumbers(
    offset_dims=(), collapsed_slice_dims=(0,), start_index_map=(0,))


def _sc_segment_sum_body(e_hbm, i_hbm, out_hbm,
                         e_v0, i_v0, e_v1, i_v1, z_v, ramp_v, m_v, b_v,
                         acc_a, acc_b, acc_sh, sem0, sem1):
    c = lax.axis_index("c")
    s = lax.axis_index("s")
    wid = c * NS + s
    base = wid * CHUNK

    def start_load(b, e_v, i_v, sem):
        off = base + b * BLK
        pltpu.make_async_copy(e_hbm.at[pl.ds(off, BLK)], e_v, sem).start()
        pltpu.make_async_copy(i_hbm.at[pl.ds(off, BLK)], i_v, sem).start()

    def wait_load(e_v, i_v, sem):
        pltpu.make_async_copy(e_hbm.at[pl.ds(base, BLK)], e_v, sem).wait()
        pltpu.make_async_copy(i_hbm.at[pl.ds(base, BLK)], i_v, sem).wait()

    start_load(0, e_v0, i_v0, sem0)
    start_load(1, e_v1, i_v1, sem1)

    zeros = jnp.zeros((L,), jnp.float32)

    @pl.loop(0, P2, step=L)
    def _zero_priv(j):
        acc_a[pl.ds(j, L)] = zeros
        acc_b[pl.ds(j, L)] = zeros

    @pl.loop(0, PS2, step=L)
    def _zero_z(j):
        z_v[pl.ds(j, L)] = zeros

    pltpu.sync_copy(z_v.at[pl.ds(0, PS2)], acc_sh.at[pl.ds(s * PS2, PS2)])
    plsc.subcore_barrier()


    def crunch(e_v, i_v):
        @pl.loop(0, BLK, step=L)
        def _vreg(j):
            sl = pl.ds(j, L)
            e = e_v[sl]
            ids = i_v[sl]
            c = plsc.cumsum(e)
            cnt, m_end = plsc.scan_count(ids)
            m_start = cnt == 0
            plsc.addupdate_scatter(acc_a, [ids], c, mask=m_end)
            plsc.addupdate_scatter(acc_b, [ids], e - c, mask=m_start)

    @pl.loop(0, NBLK, step=2)
    def _block(b):
        wait_load(e_v0, i_v0, sem0)

        @pl.when(b + 2 < NBLK)
        def _():
            start_load(b + 2, e_v0, i_v0, sem0)

        crunch(e_v0, i_v0)
        wait_load(e_v1, i_v1, sem1)

        @pl.when(b + 3 < NBLK)
        def _():
            start_load(b + 3, e_v1, i_v1, sem1)

        crunch(e_v1, i_v1)

    pltpu.sync_copy(i_hbm.at[pl.ds(base, L)], b_v)
    lo = jnp.min(b_v[...])
    pltpu.sync_copy(i_hbm.at[pl.ds(base + CHUNK - L, L)], b_v)
    hi = jnp.max(b_v[...])
    k0 = lax.div(lo, jnp.int32(CHK))

    @pl.loop(0, P // CHK + 1)
    def merge_body(i):
        off = pl.multiple_of((k0 + i) * CHK, CHK)

        @pl.when(off <= hi)
        def _():
            @pl.loop(0, CHK, step=L)
            def _sum2(j):
                o = pl.multiple_of(off + j, 8)
                ramp_v[pl.ds(j, L)] = lax.iota(jnp.int32, L) + (off + j)
                m_v[pl.ds(j, L)] = acc_a[pl.ds(o, L)] + acc_b[pl.ds(o, L)]

            pltpu.sync_copy(m_v, acc_sh.at[ramp_v], add=True)

    plsc.subcore_barrier()
    pltpu.sync_copy(acc_sh.at[pl.ds(s * PS, PS)], z_v.at[pl.ds(0, PS)])
    pltpu.sync_copy(z_v.at[pl.ds(0, PS)], out_hbm.at[pl.ds(c * P + s * PS, PS)])


def _sc_compiler_params():
    cp = pltpu.CompilerParams()
    if "needs_layout_passes" in pltpu.CompilerParams.__dataclass_fields__:
        cp = dataclasses.replace(cp, needs_layout_passes=False)
    return cp


def _sc_segment_sum(energies, ids):
    mesh = plsc.VectorSubcoreMesh(core_axis_name="c", subcore_axis_name="s")
    return pl.kernel(
        _sc_segment_sum_body,
        compiler_params=_sc_compiler_params(),
        out_type=jax.ShapeDtypeStruct((NC * P,), jnp.float32),
        mesh=mesh,
        scratch_types=[
            pltpu.VMEM((BLK,), jnp.float32),
            pltpu.VMEM((BLK,), jnp.int32),
            pltpu.VMEM((BLK,), jnp.float32),
            pltpu.VMEM((BLK,), jnp.int32),
            pltpu.VMEM((PS2,), jnp.float32),
            pltpu.VMEM((CHK,), jnp.int32),
            pltpu.VMEM((CHK,), jnp.float32),
            pltpu.VMEM((L,), jnp.int32),
            pltpu.VMEM((P2,), jnp.float32),
            pltpu.VMEM((P2,), jnp.float32),
            pltpu.VMEM_SHARED((P2,), jnp.float32),
            pltpu.SemaphoreType.DMA,
            pltpu.SemaphoreType.DMA,
        ],
    )(energies, ids)


def _tc_combine_body(p_ref, o_ref):
    o_ref[...] = p_ref[pl.ds(0, N_MOL)] + p_ref[pl.ds(P, N_MOL)]


def _tc_combine(partials_flat):
    return pl.pallas_call(
        _tc_combine_body,
        out_shape=jax.ShapeDtypeStruct((N_MOL,), jnp.float32),
    )(partials_flat)


@jax.jit
def _run(atomic_energies, ids):
    partials = _sc_segment_sum(atomic_energies, ids)
    return _tc_combine(partials)


def kernel(atomic_energies, batch):
    return _run(atomic_energies, batch.astype(jnp.int32))
